# baseline (speedup 1.0000x reference)
import functools

import numpy as np

import jax
import jax.numpy as jnp
from jax import lax
from jax.experimental import pallas as pl
from jax.experimental.pallas import tpu as pltpu
from jax.experimental.pallas import tpu_sc as plsc

N = 10000
E = 320000
D = 128
ED = 16
LD = 128

NUM_TILES = 16
EDGES_PER_TILE = E // NUM_TILES
CHUNK = 40
NUM_CHUNKS = EDGES_PER_TILE // CHUNK
GROUP = 25
NGROUPS = NUM_CHUNKS // GROUP
N_PAD = 10240
ROWS_PER_TILE = N_PAD // NUM_TILES


NB_E = 100
NB_N = 10


def _tables_body(ef_ref, z_ref, wm_ref, wc_ref, bm_ref, c_ref, a_ref, b_ref):
    i = pl.program_id(0)

    @pl.when(i < NB_E)
    def _():
        ef = ef_ref[...]
        for p in range(2):
            x = (jnp.dot(ef, wc_ref[p], preferred_element_type=jnp.float32)
                 + bm_ref[p])
            lo = lax.bitcast_convert_type(
                x[:, :LD // 2].astype(jnp.bfloat16), jnp.uint16).astype(jnp.uint32)
            hi = lax.bitcast_convert_type(
                x[:, LD // 2:].astype(jnp.bfloat16), jnp.uint16).astype(jnp.uint32)
            c_ref[p] = lo | (hi << 16)

    @pl.when(i >= NB_E)
    def _():
        zb = z_ref[...]
        for p in range(2):
            a_ref[p] = jnp.dot(zb, wm_ref[2 * p],
                               preferred_element_type=jnp.float32)
            b_ref[p] = jnp.dot(zb, wm_ref[2 * p + 1],
                               preferred_element_type=jnp.float32)


def _tables(z, e_feat, wa0, wb0, wa1, wb1, wc0, bm0, wc1, bm1):
    eblk = E // NB_E
    nblk = N // NB_N
    wm = jnp.stack([wa0, wb0, wa1, wb1])
    wc = jnp.stack([wc0, wc1])
    bm = jnp.stack([bm0, bm1])[:, None, :]
    return pl.pallas_call(
        _tables_body,
        grid=(NB_E + NB_N,),
        in_specs=[
            pl.BlockSpec((eblk, ED), lambda i: (jnp.minimum(i, NB_E - 1), 0)),
            pl.BlockSpec((nblk, D),
                         lambda i: (jnp.maximum(i - NB_E, 0), 0)),
            pl.BlockSpec((4, D, LD), lambda i: (0, 0, 0)),
            pl.BlockSpec((2, ED, LD), lambda i: (0, 0, 0)),
            pl.BlockSpec((2, 1, LD), lambda i: (0, 0, 0)),
        ],
        out_specs=[
            pl.BlockSpec((2, eblk, LD // 2),
                         lambda i: (0, jnp.minimum(i, NB_E - 1), 0)),
            pl.BlockSpec((2, nblk, LD),
                         lambda i: (0, jnp.maximum(i - NB_E, 0), 0)),
            pl.BlockSpec((2, nblk, LD),
                         lambda i: (0, jnp.maximum(i - NB_E, 0), 0)),
        ],
        out_shape=[
            jax.ShapeDtypeStruct((2, E, LD // 2), jnp.uint32),
            jax.ShapeDtypeStruct((2, N, LD), jnp.float32),
            jax.ShapeDtypeStruct((2, N, LD), jnp.float32),
        ],
    )(e_feat, z, wm, wc, bm)


def _sc_agg_body(a_hbm, b_hbm, c_hbm, src, dst, agg_out,
                 si_g, di_g, a_v0, b_v0, c_v0, a_v1, b_v1, c_v1, m_v,
                 sem_a0, sem_b0, sem_c0, sem_a1, sem_b1, sem_c1, agg_sh):
    core = lax.axis_index("c")
    tile = lax.axis_index("s")
    edge_base = tile * EDGES_PER_TILE

    def _zero_row(k, _):
        for j in range(LD // 16):
            m_v[k, pl.ds(j * 16, 16)] = jnp.zeros((16,), jnp.float32)
        return 0

    lax.fori_loop(0, CHUNK, _zero_row, 0, unroll=False)
    row0 = tile * ROWS_PER_TILE
    for done in range(0, ROWS_PER_TILE, CHUNK):
        pltpu.sync_copy(m_v, agg_sh.at[pl.ds(row0 + done, CHUNK)])
    plsc.subcore_barrier()

    bufs = ((a_v0, b_v0, c_v0, sem_a0, sem_b0, sem_c0),
            (a_v1, b_v1, c_v1, sem_a1, sem_b1, sem_c1))
    hi_mask = jnp.uint32(0xFFFF0000)

    def _issue(g, i, s):
        a_v, b_v, c_v, sem_a, sem_b, sem_c = bufs[s]
        pltpu.async_copy(a_hbm.at[core].at[si_g.at[i]], a_v, sem_a)
        pltpu.async_copy(b_hbm.at[core].at[di_g.at[i]], b_v, sem_b)
        base = edge_base + (g * GROUP + i) * CHUNK
        pltpu.async_copy(c_hbm.at[core, pl.ds(base, CHUNK)], c_v, sem_c)

    def _drain(g, i, s):
        a_v, b_v, c_v, sem_a, sem_b, sem_c = bufs[s]
        base = edge_base + (g * GROUP + i) * CHUNK
        pltpu.make_async_copy(a_hbm.at[core].at[si_g.at[i]], a_v, sem_a).wait()
        pltpu.make_async_copy(b_hbm.at[core].at[di_g.at[i]], b_v, sem_b).wait()
        pltpu.make_async_copy(
            c_hbm.at[core, pl.ds(base, CHUNK)], c_v, sem_c).wait()

        def _row(k, _):
            for j in range(LD // 32):
                ci = c_v[k, pl.ds(j * 16, 16)]
                c_lo = lax.bitcast_convert_type(ci << 16, jnp.float32)
                c_hi = lax.bitcast_convert_type(ci & hi_mask, jnp.float32)
                sl_lo = pl.ds(j * 16, 16)
                sl_hi = pl.ds(LD // 2 + j * 16, 16)
                m_v[k, sl_lo] = jnp.maximum(
                    a_v[k, sl_lo] + b_v[k, sl_lo] + c_lo, 0.0)
                m_v[k, sl_hi] = jnp.maximum(
                    a_v[k, sl_hi] + b_v[k, sl_hi] + c_hi, 0.0)
            return 0

        lax.fori_loop(0, CHUNK, _row, 0, unroll=2)

        pltpu.sync_copy(m_v, agg_sh.at[di_g.at[i]], add=True)

    def _group(g, _):
        pltpu.sync_copy(src.at[tile, g], si_g)
        pltpu.sync_copy(dst.at[tile, g], di_g)
        _issue(g, 0, 0)

        def _pair(p, _):
            i0 = 2 * p
            _issue(g, i0 + 1, 1)
            _drain(g, i0, 0)

            @pl.when(i0 + 2 < GROUP)
            def _():
                _issue(g, i0 + 2, 0)

            _drain(g, i0 + 1, 1)
            return 0

        lax.fori_loop(0, GROUP // 2, _pair, 0, unroll=False)
        if GROUP % 2 == 1:
            _drain(g, GROUP - 1, 0)
        return 0

    lax.fori_loop(0, NGROUPS, _group, 0, unroll=False)
    plsc.subcore_barrier()
    pltpu.sync_copy(agg_sh.at[pl.ds(row0, ROWS_PER_TILE)],
                    agg_out.at[core, pl.ds(row0, ROWS_PER_TILE)])


def _sc_agg(a, b, c, src, dst):
    mesh = plsc.VectorSubcoreMesh(core_axis_name="c", subcore_axis_name="s")
    fn = pl.kernel(
        _sc_agg_body,
        out_type=jax.ShapeDtypeStruct((2, N_PAD, LD), jnp.float32),
        mesh=mesh,
        scratch_types=[
            pltpu.VMEM((GROUP, CHUNK), jnp.int32),
            pltpu.VMEM((GROUP, CHUNK), jnp.int32),
            pltpu.VMEM((CHUNK, LD), jnp.float32),
            pltpu.VMEM((CHUNK, LD), jnp.float32),
            pltpu.VMEM((CHUNK, LD // 2), jnp.uint32),
            pltpu.VMEM((CHUNK, LD), jnp.float32),
            pltpu.VMEM((CHUNK, LD), jnp.float32),
            pltpu.VMEM((CHUNK, LD // 2), jnp.uint32),
            pltpu.VMEM((CHUNK, LD), jnp.float32),
            pltpu.SemaphoreType.DMA,
            pltpu.SemaphoreType.DMA,
            pltpu.SemaphoreType.DMA,
            pltpu.SemaphoreType.DMA,
            pltpu.SemaphoreType.DMA,
            pltpu.SemaphoreType.DMA,
            pltpu.VMEM_SHARED((N_PAD, LD), jnp.float32),
        ],
    )
    src4 = src.reshape(NUM_TILES, NGROUPS, GROUP, CHUNK)
    dst4 = dst.reshape(NUM_TILES, NGROUPS, GROUP, CHUNK)
    return fn(a, b, c, src4, dst4)


def _update_body(enc_ref, agg_ref, w_ref, b_ref, coef_ref, out_ref):
    encb = enc_ref[...]
    x0 = (jnp.dot(encb, w_ref[0], preferred_element_type=jnp.float32)
          + jnp.dot(agg_ref[0], w_ref[1], preferred_element_type=jnp.float32)
          + b_ref[0])
    x1 = (jnp.dot(encb, w_ref[2], preferred_element_type=jnp.float32)
          + jnp.dot(agg_ref[1], w_ref[3], preferred_element_type=jnp.float32)
          + b_ref[1])
    out_ref[...] = (coef_ref[0] * jnp.maximum(x0, 0.0)
                    + coef_ref[1] * jnp.maximum(x1, 0.0))


def _update(enc, agg, wE0, wA0, bu0, wE1, wA1, bu1, coef):
    nb = 10
    blk = N // nb
    w = jnp.stack([wE0, wA0, wE1, wA1])
    b = jnp.stack([bu0, bu1])[:, None, :]
    return pl.pallas_call(
        _update_body,
        grid=(nb,),
        in_specs=[
            pl.BlockSpec((blk, D), lambda i: (i, 0)),
            pl.BlockSpec((2, blk, LD), lambda i: (0, i, 0)),
            pl.BlockSpec((4, D, LD), lambda i: (0, 0, 0)),
            pl.BlockSpec((2, 1, LD), lambda i: (0, 0, 0)),
            pl.BlockSpec(memory_space=pltpu.SMEM),
        ],
        out_specs=pl.BlockSpec((blk, LD), lambda i: (i, 0)),
        out_shape=jax.ShapeDtypeStruct((N, LD), jnp.float32),
    )(enc, agg, w, b, coef)


def kernel(z, e_feat, adj, enc, W_msg_0, b_msg_0, W_upd_0, b_upd_0,
           W_msg_1, b_msg_1, W_upd_1, b_upd_1, coef):
    src = adj[0]
    dst = adj[1]

    c, a, b = _tables(
        z, e_feat, W_msg_0[:D], W_msg_0[D:2 * D], W_msg_1[:D],
        W_msg_1[D:2 * D], W_msg_0[2 * D:], b_msg_0, W_msg_1[2 * D:], b_msg_1)

    agg = _sc_agg(a, b, c, src, dst)

    return _update(enc, agg,
                   W_upd_0[:D], W_upd_0[D:], b_upd_0,
                   W_upd_1[:D], W_upd_1[D:], b_upd_1, coef)


# speedup vs baseline: 1.7925x; 1.7925x over previous
import jax
import jax.numpy as jnp
from jax import lax
from jax.experimental import pallas as pl
from jax.experimental.pallas import tpu as pltpu
from jax.experimental.pallas import tpu_sc as plsc

N = 10000
E = 320000
D = 128
ED = 16
LD = 128

NUM_TILES = 16
EDGES_PER_TILE = E // NUM_TILES
CHUNK = 40
NUM_CHUNKS = EDGES_PER_TILE // CHUNK
GROUP = 25
NGROUPS = NUM_CHUNKS // GROUP
N_PAD = 10240
ROWS_PER_TILE = N_PAD // NUM_TILES


NB_E = 100
NB_N = 10


def _tables_body(ef_ref, z_ref, wm_ref, wc_ref, bm_ref, c_ref, a_ref, b_ref):
    i = pl.program_id(0)

    @pl.when(i < NB_E)
    def _():
        ef = ef_ref[...]
        for p in range(2):
            x = (jnp.dot(ef, wc_ref[p], preferred_element_type=jnp.float32)
                 + bm_ref[p])
            lo = lax.bitcast_convert_type(
                x[:, :LD // 2].astype(jnp.bfloat16), jnp.uint16).astype(jnp.uint32)
            hi = lax.bitcast_convert_type(
                x[:, LD // 2:].astype(jnp.bfloat16), jnp.uint16).astype(jnp.uint32)
            c_ref[p] = lo | (hi << 16)

    @pl.when(i >= NB_E)
    def _():
        zb = z_ref[...]
        for p in range(2):
            a_ref[p] = jnp.dot(zb, wm_ref[2 * p],
                               preferred_element_type=jnp.float32)
            b_ref[p] = jnp.dot(zb, wm_ref[2 * p + 1],
                               preferred_element_type=jnp.float32)


def _tables(z, e_feat, wa0, wb0, wa1, wb1, wc0, bm0, wc1, bm1):
    eblk = E // NB_E
    nblk = N // NB_N
    wm = jnp.stack([wa0, wb0, wa1, wb1])
    wc = jnp.stack([wc0, wc1])
    bm = jnp.stack([bm0, bm1])[:, None, :]
    return pl.pallas_call(
        _tables_body,
        grid=(NB_E + NB_N,),
        in_specs=[
            pl.BlockSpec((eblk, ED), lambda i: (jnp.minimum(i, NB_E - 1), 0)),
            pl.BlockSpec((nblk, D),
                         lambda i: (jnp.maximum(i - NB_E, 0), 0)),
            pl.BlockSpec((4, D, LD), lambda i: (0, 0, 0)),
            pl.BlockSpec((2, ED, LD), lambda i: (0, 0, 0)),
            pl.BlockSpec((2, 1, LD), lambda i: (0, 0, 0)),
        ],
        out_specs=[
            pl.BlockSpec((2, eblk, LD // 2),
                         lambda i: (0, jnp.minimum(i, NB_E - 1), 0)),
            pl.BlockSpec((2, nblk, LD),
                         lambda i: (0, jnp.maximum(i - NB_E, 0), 0)),
            pl.BlockSpec((2, nblk, LD),
                         lambda i: (0, jnp.maximum(i - NB_E, 0), 0)),
        ],
        out_shape=[
            jax.ShapeDtypeStruct((2, E, LD // 2), jnp.uint32),
            jax.ShapeDtypeStruct((2, N, LD), jnp.float32),
            jax.ShapeDtypeStruct((2, N, LD), jnp.float32),
        ],
    )(e_feat, z, wm, wc, bm)


def _sc_agg_body(a_hbm, b_hbm, c_hbm, src, dst, agg_out,
                 si_g, di_g, a_v0, b_v0, c_v0, a_v1, b_v1, c_v1, m_v,
                 sem_a0, sem_b0, sem_c0, sem_a1, sem_b1, sem_c1, agg_sh):
    core = lax.axis_index("c")
    tile = lax.axis_index("s")
    edge_base = tile * EDGES_PER_TILE

    def _zero_row(k, _):
        for j in range(LD // 16):
            m_v[k, pl.ds(j * 16, 16)] = jnp.zeros((16,), jnp.float32)
        return 0

    lax.fori_loop(0, CHUNK, _zero_row, 0, unroll=False)
    row0 = tile * ROWS_PER_TILE
    for done in range(0, ROWS_PER_TILE, CHUNK):
        pltpu.sync_copy(m_v, agg_sh.at[pl.ds(row0 + done, CHUNK)])
    plsc.subcore_barrier()

    bufs = ((a_v0, b_v0, c_v0, sem_a0, sem_b0, sem_c0),
            (a_v1, b_v1, c_v1, sem_a1, sem_b1, sem_c1))
    hi_mask = jnp.uint32(0xFFFF0000)

    def _issue(g, i, s):
        a_v, b_v, c_v, sem_a, sem_b, sem_c = bufs[s]
        pltpu.async_copy(a_hbm.at[core].at[si_g.at[i]], a_v, sem_a)
        pltpu.async_copy(b_hbm.at[core].at[di_g.at[i]], b_v, sem_b)
        base = edge_base + (g * GROUP + i) * CHUNK
        pltpu.async_copy(c_hbm.at[core, pl.ds(base, CHUNK)], c_v, sem_c)

    def _drain(g, i, s):
        a_v, b_v, c_v, sem_a, sem_b, sem_c = bufs[s]
        base = edge_base + (g * GROUP + i) * CHUNK
        pltpu.make_async_copy(a_hbm.at[core].at[si_g.at[i]], a_v, sem_a).wait()
        pltpu.make_async_copy(b_hbm.at[core].at[di_g.at[i]], b_v, sem_b).wait()
        pltpu.make_async_copy(
            c_hbm.at[core, pl.ds(base, CHUNK)], c_v, sem_c).wait()

        def _row(k, _):
            for j in range(LD // 32):
                ci = c_v[k, pl.ds(j * 16, 16)]
                c_lo = lax.bitcast_convert_type(ci << 16, jnp.float32)
                c_hi = lax.bitcast_convert_type(ci & hi_mask, jnp.float32)
                sl_lo = pl.ds(j * 16, 16)
                sl_hi = pl.ds(LD // 2 + j * 16, 16)
                m_v[k, sl_lo] = jnp.maximum(
                    a_v[k, sl_lo] + b_v[k, sl_lo] + c_lo, 0.0)
                m_v[k, sl_hi] = jnp.maximum(
                    a_v[k, sl_hi] + b_v[k, sl_hi] + c_hi, 0.0)
            return 0

        lax.fori_loop(0, CHUNK, _row, 0, unroll=False)

        pltpu.sync_copy(m_v, agg_sh.at[di_g.at[i]], add=True)

    def _group(g, _):
        pltpu.sync_copy(src.at[tile, g], si_g)
        pltpu.sync_copy(dst.at[tile, g], di_g)
        _issue(g, 0, 0)

        def _pair(p, _):
            i0 = 2 * p
            _issue(g, i0 + 1, 1)
            _drain(g, i0, 0)

            @pl.when(i0 + 2 < GROUP)
            def _():
                _issue(g, i0 + 2, 0)

            _drain(g, i0 + 1, 1)
            return 0

        lax.fori_loop(0, GROUP // 2, _pair, 0, unroll=False)
        if GROUP % 2 == 1:
            _drain(g, GROUP - 1, 0)
        return 0

    lax.fori_loop(0, NGROUPS, _group, 0, unroll=False)
    plsc.subcore_barrier()
    pltpu.sync_copy(agg_sh.at[pl.ds(row0, ROWS_PER_TILE)],
                    agg_out.at[core, pl.ds(row0, ROWS_PER_TILE)])


def _sc_agg(a, b, c, src, dst):
    mesh = plsc.VectorSubcoreMesh(core_axis_name="c", subcore_axis_name="s")
    fn = pl.kernel(
        _sc_agg_body,
        out_type=jax.ShapeDtypeStruct((2, N_PAD, LD), jnp.float32),
        mesh=mesh,
        scratch_types=[
            pltpu.VMEM((GROUP, CHUNK), jnp.int32),
            pltpu.VMEM((GROUP, CHUNK), jnp.int32),
            pltpu.VMEM((CHUNK, LD), jnp.float32),
            pltpu.VMEM((CHUNK, LD), jnp.float32),
            pltpu.VMEM((CHUNK, LD // 2), jnp.uint32),
            pltpu.VMEM((CHUNK, LD), jnp.float32),
            pltpu.VMEM((CHUNK, LD), jnp.float32),
            pltpu.VMEM((CHUNK, LD // 2), jnp.uint32),
            pltpu.VMEM((CHUNK, LD), jnp.float32),
            pltpu.SemaphoreType.DMA,
            pltpu.SemaphoreType.DMA,
            pltpu.SemaphoreType.DMA,
            pltpu.SemaphoreType.DMA,
            pltpu.SemaphoreType.DMA,
            pltpu.SemaphoreType.DMA,
            pltpu.VMEM_SHARED((N_PAD, LD), jnp.float32),
        ],
    )
    src4 = src.reshape(NUM_TILES, NGROUPS, GROUP, CHUNK)
    dst4 = dst.reshape(NUM_TILES, NGROUPS, GROUP, CHUNK)
    return fn(a, b, c, src4, dst4)


def _update_body(enc_ref, agg_ref, w_ref, b_ref, coef_ref, out_ref):
    encb = enc_ref[...]
    x0 = (jnp.dot(encb, w_ref[0], preferred_element_type=jnp.float32)
          + jnp.dot(agg_ref[0], w_ref[1], preferred_element_type=jnp.float32)
          + b_ref[0])
    x1 = (jnp.dot(encb, w_ref[2], preferred_element_type=jnp.float32)
          + jnp.dot(agg_ref[1], w_ref[3], preferred_element_type=jnp.float32)
          + b_ref[1])
    out_ref[...] = (coef_ref[0] * jnp.maximum(x0, 0.0)
                    + coef_ref[1] * jnp.maximum(x1, 0.0))


def _update(enc, agg, wE0, wA0, bu0, wE1, wA1, bu1, coef):
    nb = 10
    blk = N // nb
    w = jnp.stack([wE0, wA0, wE1, wA1])
    b = jnp.stack([bu0, bu1])[:, None, :]
    return pl.pallas_call(
        _update_body,
        grid=(nb,),
        in_specs=[
            pl.BlockSpec((blk, D), lambda i: (i, 0)),
            pl.BlockSpec((2, blk, LD), lambda i: (0, i, 0)),
            pl.BlockSpec((4, D, LD), lambda i: (0, 0, 0)),
            pl.BlockSpec((2, 1, LD), lambda i: (0, 0, 0)),
            pl.BlockSpec(memory_space=pltpu.SMEM),
        ],
        out_specs=pl.BlockSpec((blk, LD), lambda i: (i, 0)),
        out_shape=jax.ShapeDtypeStruct((N, LD), jnp.float32),
    )(enc, agg, w, b, coef)


def kernel(z, e_feat, adj, enc, W_msg_0, b_msg_0, W_upd_0, b_upd_0,
           W_msg_1, b_msg_1, W_upd_1, b_upd_1, coef):
    src = adj[0]
    dst = adj[1]

    c, a, b = _tables(
        z, e_feat, W_msg_0[:D], W_msg_0[D:2 * D], W_msg_1[:D],
        W_msg_1[D:2 * D], W_msg_0[2 * D:], b_msg_0, W_msg_1[2 * D:], b_msg_1)

    agg = _sc_agg(a, b, c, src, dst)

    return _update(enc, agg,
                   W_upd_0[:D], W_upd_0[D:], b_upd_0,
                   W_upd_1[:D], W_upd_1[D:], b_upd_1, coef)
